# baseline (speedup 1.0000x reference)
import functools

import jax
import jax.numpy as jnp
from jax import lax
from jax.experimental import pallas as pl
from jax.experimental.pallas import tpu as pltpu
from jax.experimental.pallas import tpu_sc as plsc

D = 2048
E = 16
I = 1024
S = 2048
K = 2
SI = 2 * I

BM = 256
RPT = S * K + E * BM
NB = RPT // BM

NRB = 2
NIC = 8
BMS = S // NRB
CSI = SI // NIC


def _route(indices, weights):
    flat_e = indices.reshape(-1).astype(jnp.int32)
    order = jnp.argsort(flat_e)
    sorted_e = flat_e[order]
    counts = jnp.bincount(flat_e, length=E)
    padded = ((counts + BM - 1) // BM) * BM
    ucum = jnp.cumsum(counts)
    pcum = jnp.cumsum(padded)
    i = jnp.arange(S * K)
    pos_sorted = (pcum[sorted_e] - padded[sorted_e]
                  + (i - (ucum[sorted_e] - counts[sorted_e]))).astype(jnp.int32)
    token_pad = (jnp.arange(RPT, dtype=jnp.int32) % S).at[pos_sorted].set(
        (order // K).astype(jnp.int32))
    w_pad = jnp.zeros((RPT,), jnp.float32).at[pos_sorted].set(
        weights.reshape(-1)[order])
    slot_pos = jnp.zeros((S * K,), jnp.int32).at[order].set(pos_sorted)
    pos = slot_pos.reshape(S, K).T
    bs = jnp.arange(NB, dtype=jnp.int32) * BM
    be = jnp.searchsorted(pcum, bs, side='right').astype(jnp.int32)
    active = bs < pcum[-1]
    be = jnp.where(active, be, sorted_e[-1])
    xsb = jnp.minimum(jnp.arange(NB, dtype=jnp.int32),
                      (pcum[-1] // BM - 1).astype(jnp.int32))
    meta = jnp.stack([be, active.astype(jnp.int32), xsb])
    return token_pad, w_pad, pos, meta


def _group_body(meta_ref, xs_ref, w1_ref, w3_ref, w2_ref, wr_ref, ys_ref):
    b = pl.program_id(0)

    @pl.when(meta_ref[1, b] == 1)
    def _():
        xb = xs_ref[...].astype(jnp.bfloat16)
        w1 = w1_ref[0].astype(jnp.bfloat16)
        w3 = w3_ref[0].astype(jnp.bfloat16)
        w2 = w2_ref[0].astype(jnp.bfloat16)
        a = lax.dot_general(xb, w1, (((1,), (1,)), ((), ())),
                            preferred_element_type=jnp.float32)
        c = lax.dot_general(xb, w3, (((1,), (1,)), ((), ())),
                            preferred_element_type=jnp.float32)
        h = (jax.nn.silu(a) * c).astype(jnp.bfloat16)
        out = lax.dot_general(h, w2, (((1,), (1,)), ((), ())),
                              preferred_element_type=jnp.float32)
        ys_ref[...] = out * wr_ref[0, 0, :][:, None]


def _grouped_mlp(meta, xs, W1, W3, W2, w_pad):
    w3d = w_pad.reshape(NB, 1, BM)
    grid_spec = pltpu.PrefetchScalarGridSpec(
        num_scalar_prefetch=1,
        grid=(NB,),
        in_specs=[
            pl.BlockSpec((BM, D), lambda b, m: (m[2, b], 0)),
            pl.BlockSpec((1, I, D), lambda b, m: (m[0, b], 0, 0)),
            pl.BlockSpec((1, I, D), lambda b, m: (m[0, b], 0, 0)),
            pl.BlockSpec((1, D, I), lambda b, m: (m[0, b], 0, 0)),
            pl.BlockSpec((1, 1, BM), lambda b, m: (b, 0, 0)),
        ],
        out_specs=pl.BlockSpec((BM, D), lambda b, m: (b, 0)),
    )
    return pl.pallas_call(
        _group_body,
        grid_spec=grid_spec,
        out_shape=jax.ShapeDtypeStruct((RPT, D), jnp.float32),
        compiler_params=pltpu.CompilerParams(
            dimension_semantics=("arbitrary",),
            vmem_limit_bytes=120 * 1024 * 1024,
        ),
    )(meta, xs, W1, W3, W2, w3d)


def _shared_body(x_ref, w1_ref, w3_ref, w2_ref, z_ref):
    ic = pl.program_id(1)
    xb = x_ref[...].astype(jnp.bfloat16)
    w1 = w1_ref[...].astype(jnp.bfloat16)
    w3 = w3_ref[...].astype(jnp.bfloat16)
    w2 = w2_ref[...].astype(jnp.bfloat16)
    a = lax.dot_general(xb, w1, (((1,), (1,)), ((), ())),
                        preferred_element_type=jnp.float32)
    c = lax.dot_general(xb, w3, (((1,), (1,)), ((), ())),
                        preferred_element_type=jnp.float32)
    h = (jax.nn.silu(a) * c).astype(jnp.bfloat16)
    out = lax.dot_general(h, w2, (((1,), (1,)), ((), ())),
                          preferred_element_type=jnp.float32)

    @pl.when(ic == 0)
    def _():
        z_ref[...] = out

    @pl.when(ic != 0)
    def _():
        z_ref[...] += out


def _shared_mlp(xf, Ws1, Ws3, Ws2):
    return pl.pallas_call(
        _shared_body,
        grid=(NRB, NIC),
        in_specs=[
            pl.BlockSpec((BMS, D), lambda r, ic: (r, 0)),
            pl.BlockSpec((CSI, D), lambda r, ic: (ic, 0)),
            pl.BlockSpec((CSI, D), lambda r, ic: (ic, 0)),
            pl.BlockSpec((D, CSI), lambda r, ic: (0, ic)),
        ],
        out_specs=pl.BlockSpec((BMS, D), lambda r, ic: (r, 0)),
        out_shape=jax.ShapeDtypeStruct((S, D), jnp.float32),
        compiler_params=pltpu.CompilerParams(
            dimension_semantics=("arbitrary", "arbitrary"),
            vmem_limit_bytes=120 * 1024 * 1024,
        ),
    )(xf, Ws1, Ws3, Ws2)


NC, NS = 2, 16
NW = NC * NS
RPW = RPT // NW
GCH = 32


def _make_row_gather(n_rows, gch):
    rpw = n_rows // NW
    nch = rpw // gch

    def body(src_hbm, idx_hbm, out_hbm, idx_v,
             rows0, rows1, rows2, g0, g1, g2, o0, o1, o2):
        wid = lax.axis_index("s") * NC + lax.axis_index("c")
        base = wid * rpw
        pltpu.sync_copy(idx_hbm.at[pl.ds(base, rpw)], idx_v)
        bufs = (rows0, rows1, rows2)
        gsem = (g0, g1, g2)
        osem = (o0, o1, o2)

        def fire(c):
            return pltpu.async_copy(
                src_hbm.at[idx_v.at[pl.ds(c * gch, gch)]],
                bufs[c % 3], gsem[c % 3])

        gd = [None] * nch
        od = [None] * nch
        gd[0] = fire(0)
        if nch > 1:
            gd[1] = fire(1)
        for c in range(nch):
            gd[c].wait()
            od[c] = pltpu.async_copy(
                bufs[c % 3], out_hbm.at[pl.ds(base + c * gch, gch)],
                osem[c % 3])
            if c + 2 < nch:
                if c - 1 >= 0:
                    od[c - 1].wait()
                gd[c + 2] = fire(c + 2)
        for c in range(max(0, nch - 3), nch):
            od[c].wait()

    mesh = plsc.VectorSubcoreMesh(core_axis_name="c", subcore_axis_name="s")
    return pl.kernel(
        body,
        out_type=jax.ShapeDtypeStruct((n_rows, D), jnp.float32),
        mesh=mesh,
        scratch_types=[
            pltpu.VMEM((rpw,), jnp.int32),
            pltpu.VMEM((gch, D), jnp.float32),
            pltpu.VMEM((gch, D), jnp.float32),
            pltpu.VMEM((gch, D), jnp.float32),
            pltpu.SemaphoreType.DMA,
            pltpu.SemaphoreType.DMA,
            pltpu.SemaphoreType.DMA,
            pltpu.SemaphoreType.DMA,
            pltpu.SemaphoreType.DMA,
            pltpu.SemaphoreType.DMA,
        ],
    )


def _gather(xf, token_pad):
    return _make_row_gather(RPT, 16)(xf, token_pad)


BMC = 256


def _add3_body(g0_ref, g1_ref, z_ref, y_ref):
    y_ref[...] = g0_ref[0] + g1_ref[0] + z_ref[...]


def _combine(ys, pos, z):
    g = _make_row_gather(K * S, 16)(ys, pos.reshape(-1))
    g3 = g.reshape(K, S, D)
    return pl.pallas_call(
        _add3_body,
        grid=(S // BMC,),
        in_specs=[
            pl.BlockSpec((1, BMC, D), lambda r: (0, r, 0)),
            pl.BlockSpec((1, BMC, D), lambda r: (1, r, 0)),
            pl.BlockSpec((BMC, D), lambda r: (r, 0)),
        ],
        out_specs=pl.BlockSpec((BMC, D), lambda r: (r, 0)),
        out_shape=jax.ShapeDtypeStruct((S, D), jnp.float32),
        compiler_params=pltpu.CompilerParams(
            dimension_semantics=("arbitrary",),
        ),
    )(g3, g3, z)


def kernel(x, weights, indices, W1, W2, W3, Ws1, Ws2, Ws3):
    xf = x.reshape(-1, D)
    token_pad, w_pad, pos, meta = _route(indices, weights)
    xs = _gather(xf, token_pad)
    z = _shared_mlp(xf, Ws1, Ws3, Ws2)
    ys = _grouped_mlp(meta, xs, W1, W3, W2, w_pad)
    y = _combine(ys, pos, z)
    return y.reshape(x.shape)


# speedup vs baseline: 1.0119x; 1.0119x over previous
import functools

import jax
import jax.numpy as jnp
from jax import lax
from jax.experimental import pallas as pl
from jax.experimental.pallas import tpu as pltpu
from jax.experimental.pallas import tpu_sc as plsc

D = 2048
E = 16
I = 1024
S = 2048
K = 2
SI = 2 * I

BM = 256
RPT = S * K + E * BM
NB = RPT // BM

NRB = 2
NIC = 4
BMS = S // NRB
CSI = SI // NIC


def _route(indices, weights):
    flat_e = indices.reshape(-1).astype(jnp.int32)
    order = jnp.argsort(flat_e)
    sorted_e = flat_e[order]
    counts = jnp.bincount(flat_e, length=E)
    padded = ((counts + BM - 1) // BM) * BM
    ucum = jnp.cumsum(counts)
    pcum = jnp.cumsum(padded)
    i = jnp.arange(S * K)
    pos_sorted = (pcum[sorted_e] - padded[sorted_e]
                  + (i - (ucum[sorted_e] - counts[sorted_e]))).astype(jnp.int32)
    token_pad = (jnp.arange(RPT, dtype=jnp.int32) % S).at[pos_sorted].set(
        (order // K).astype(jnp.int32))
    w_pad = jnp.zeros((RPT,), jnp.float32).at[pos_sorted].set(
        weights.reshape(-1)[order])
    slot_pos = jnp.zeros((S * K,), jnp.int32).at[order].set(pos_sorted)
    pos = slot_pos.reshape(S, K).T
    bs = jnp.arange(NB, dtype=jnp.int32) * BM
    be = jnp.searchsorted(pcum, bs, side='right').astype(jnp.int32)
    active = bs < pcum[-1]
    be = jnp.where(active, be, sorted_e[-1])
    xsb = jnp.minimum(jnp.arange(NB, dtype=jnp.int32),
                      (pcum[-1] // BM - 1).astype(jnp.int32))
    meta = jnp.stack([be, active.astype(jnp.int32), xsb])
    return token_pad, w_pad, pos, meta


def _group_body(meta_ref, xs_ref, w1_ref, w3_ref, w2_ref, wr_ref, ys_ref):
    b = pl.program_id(0)

    @pl.when(meta_ref[1, b] == 1)
    def _():
        xb = xs_ref[...].astype(jnp.bfloat16)
        w1 = w1_ref[0].astype(jnp.bfloat16)
        w3 = w3_ref[0].astype(jnp.bfloat16)
        w2 = w2_ref[0].astype(jnp.bfloat16)
        a = lax.dot_general(xb, w1, (((1,), (1,)), ((), ())),
                            preferred_element_type=jnp.float32)
        c = lax.dot_general(xb, w3, (((1,), (1,)), ((), ())),
                            preferred_element_type=jnp.float32)
        h = (jax.nn.silu(a) * c).astype(jnp.bfloat16)
        out = lax.dot_general(h, w2, (((1,), (1,)), ((), ())),
                              preferred_element_type=jnp.float32)
        ys_ref[...] = out * wr_ref[0, 0, :][:, None]


def _grouped_mlp(meta, xs, W1, W3, W2, w_pad):
    w3d = w_pad.reshape(NB, 1, BM)
    grid_spec = pltpu.PrefetchScalarGridSpec(
        num_scalar_prefetch=1,
        grid=(NB,),
        in_specs=[
            pl.BlockSpec((BM, D), lambda b, m: (m[2, b], 0)),
            pl.BlockSpec((1, I, D), lambda b, m: (m[0, b], 0, 0)),
            pl.BlockSpec((1, I, D), lambda b, m: (m[0, b], 0, 0)),
            pl.BlockSpec((1, D, I), lambda b, m: (m[0, b], 0, 0)),
            pl.BlockSpec((1, 1, BM), lambda b, m: (b, 0, 0)),
        ],
        out_specs=pl.BlockSpec((BM, D), lambda b, m: (b, 0)),
    )
    return pl.pallas_call(
        _group_body,
        grid_spec=grid_spec,
        out_shape=jax.ShapeDtypeStruct((RPT, D), jnp.float32),
        compiler_params=pltpu.CompilerParams(
            dimension_semantics=("arbitrary",),
            vmem_limit_bytes=120 * 1024 * 1024,
        ),
    )(meta, xs, W1, W3, W2, w3d)


def _shared_body(x_ref, w1_ref, w3_ref, w2_ref, z_ref):
    ic = pl.program_id(1)
    xb = x_ref[...].astype(jnp.bfloat16)
    w1 = w1_ref[...].astype(jnp.bfloat16)
    w3 = w3_ref[...].astype(jnp.bfloat16)
    w2 = w2_ref[...].astype(jnp.bfloat16)
    a = lax.dot_general(xb, w1, (((1,), (1,)), ((), ())),
                        preferred_element_type=jnp.float32)
    c = lax.dot_general(xb, w3, (((1,), (1,)), ((), ())),
                        preferred_element_type=jnp.float32)
    h = (jax.nn.silu(a) * c).astype(jnp.bfloat16)
    out = lax.dot_general(h, w2, (((1,), (1,)), ((), ())),
                          preferred_element_type=jnp.float32)

    @pl.when(ic == 0)
    def _():
        z_ref[...] = out

    @pl.when(ic != 0)
    def _():
        z_ref[...] += out


def _shared_mlp(xf, Ws1, Ws3, Ws2):
    return pl.pallas_call(
        _shared_body,
        grid=(NRB, NIC),
        in_specs=[
            pl.BlockSpec((BMS, D), lambda r, ic: (r, 0)),
            pl.BlockSpec((CSI, D), lambda r, ic: (ic, 0)),
            pl.BlockSpec((CSI, D), lambda r, ic: (ic, 0)),
            pl.BlockSpec((D, CSI), lambda r, ic: (0, ic)),
        ],
        out_specs=pl.BlockSpec((BMS, D), lambda r, ic: (r, 0)),
        out_shape=jax.ShapeDtypeStruct((S, D), jnp.float32),
        compiler_params=pltpu.CompilerParams(
            dimension_semantics=("arbitrary", "arbitrary"),
            vmem_limit_bytes=120 * 1024 * 1024,
        ),
    )(xf, Ws1, Ws3, Ws2)


NC, NS = 2, 16
NW = NC * NS
RPW = RPT // NW
GCH = 32


def _make_row_gather(n_rows, gch):
    rpw = n_rows // NW
    nch = rpw // gch

    def body(src_hbm, idx_hbm, out_hbm, idx_v,
             rows0, rows1, rows2, g0, g1, g2, o0, o1, o2):
        wid = lax.axis_index("s") * NC + lax.axis_index("c")
        base = wid * rpw
        pltpu.sync_copy(idx_hbm.at[pl.ds(base, rpw)], idx_v)
        bufs = (rows0, rows1, rows2)
        gsem = (g0, g1, g2)
        osem = (o0, o1, o2)

        def fire(c):
            return pltpu.async_copy(
                src_hbm.at[idx_v.at[pl.ds(c * gch, gch)]],
                bufs[c % 3], gsem[c % 3])

        gd = [None] * nch
        od = [None] * nch
        gd[0] = fire(0)
        if nch > 1:
            gd[1] = fire(1)
        for c in range(nch):
            gd[c].wait()
            od[c] = pltpu.async_copy(
                bufs[c % 3], out_hbm.at[pl.ds(base + c * gch, gch)],
                osem[c % 3])
            if c + 2 < nch:
                if c - 1 >= 0:
                    od[c - 1].wait()
                gd[c + 2] = fire(c + 2)
        for c in range(max(0, nch - 3), nch):
            od[c].wait()

    mesh = plsc.VectorSubcoreMesh(core_axis_name="c", subcore_axis_name="s")
    return pl.kernel(
        body,
        out_type=jax.ShapeDtypeStruct((n_rows, D), jnp.float32),
        mesh=mesh,
        scratch_types=[
            pltpu.VMEM((rpw,), jnp.int32),
            pltpu.VMEM((gch, D), jnp.float32),
            pltpu.VMEM((gch, D), jnp.float32),
            pltpu.VMEM((gch, D), jnp.float32),
            pltpu.SemaphoreType.DMA,
            pltpu.SemaphoreType.DMA,
            pltpu.SemaphoreType.DMA,
            pltpu.SemaphoreType.DMA,
            pltpu.SemaphoreType.DMA,
            pltpu.SemaphoreType.DMA,
        ],
    )


def _gather(xf, token_pad):
    return _make_row_gather(RPT, 16)(xf, token_pad)


BMC = 256


def _add3_body(g0_ref, g1_ref, z_ref, y_ref):
    y_ref[...] = g0_ref[0] + g1_ref[0] + z_ref[...]


def _combine(ys, pos, z):
    g = _make_row_gather(K * S, 16)(ys, pos.reshape(-1))
    g3 = g.reshape(K, S, D)
    return pl.pallas_call(
        _add3_body,
        grid=(S // BMC,),
        in_specs=[
            pl.BlockSpec((1, BMC, D), lambda r: (0, r, 0)),
            pl.BlockSpec((1, BMC, D), lambda r: (1, r, 0)),
            pl.BlockSpec((BMC, D), lambda r: (r, 0)),
        ],
        out_specs=pl.BlockSpec((BMC, D), lambda r: (r, 0)),
        out_shape=jax.ShapeDtypeStruct((S, D), jnp.float32),
        compiler_params=pltpu.CompilerParams(
            dimension_semantics=("arbitrary",),
        ),
    )(g3, g3, z)


def kernel(x, weights, indices, W1, W2, W3, Ws1, Ws2, Ws3):
    xf = x.reshape(-1, D)
    token_pad, w_pad, pos, meta = _route(indices, weights)
    xs = _gather(xf, token_pad)
    z = _shared_mlp(xf, Ws1, Ws3, Ws2)
    ys = _grouped_mlp(meta, xs, W1, W3, W2, w_pad)
    y = _combine(ys, pos, z)
    return y.reshape(x.shape)


# speedup vs baseline: 1.0252x; 1.0131x over previous
import functools

import jax
import jax.numpy as jnp
from jax import lax
from jax.experimental import pallas as pl
from jax.experimental.pallas import tpu as pltpu
from jax.experimental.pallas import tpu_sc as plsc

D = 2048
E = 16
I = 1024
S = 2048
K = 2
SI = 2 * I

BM = 256
RPT = S * K + E * BM
NB = RPT // BM

NRB = 2
NIC = 4
BMS = S // NRB
CSI = SI // NIC


def _route(indices, weights):
    flat_e = indices.reshape(-1).astype(jnp.int32)
    order = jnp.argsort(flat_e)
    sorted_e = flat_e[order]
    counts = jnp.bincount(flat_e, length=E)
    padded = ((counts + BM - 1) // BM) * BM
    ucum = jnp.cumsum(counts)
    pcum = jnp.cumsum(padded)
    i = jnp.arange(S * K)
    pos_sorted = (pcum[sorted_e] - padded[sorted_e]
                  + (i - (ucum[sorted_e] - counts[sorted_e]))).astype(jnp.int32)
    token_pad = (jnp.arange(RPT, dtype=jnp.int32) % S).at[pos_sorted].set(
        (order // K).astype(jnp.int32))
    w_pad = jnp.zeros((RPT,), jnp.float32).at[pos_sorted].set(
        weights.reshape(-1)[order])
    slot_pos = jnp.zeros((S * K,), jnp.int32).at[order].set(pos_sorted)
    pos = slot_pos.reshape(S, K).T
    bs = jnp.arange(NB, dtype=jnp.int32) * BM
    be = jnp.searchsorted(pcum, bs, side='right').astype(jnp.int32)
    active = bs < pcum[-1]
    be = jnp.where(active, be, sorted_e[-1])
    xsb = jnp.minimum(jnp.arange(NB, dtype=jnp.int32),
                      (pcum[-1] // BM - 1).astype(jnp.int32))
    meta = jnp.stack([be, active.astype(jnp.int32), xsb])
    tot16 = jnp.full((16,), pcum[-1], jnp.int32)
    return token_pad, w_pad, pos, meta, tot16


def _group_body(meta_ref, xs_ref, w1_ref, w3_ref, w2_ref, wr_ref, ys_ref):
    b = pl.program_id(0)

    @pl.when(meta_ref[1, b] == 1)
    def _():
        xb = xs_ref[...].astype(jnp.bfloat16)
        w1 = w1_ref[0].astype(jnp.bfloat16)
        w3 = w3_ref[0].astype(jnp.bfloat16)
        w2 = w2_ref[0].astype(jnp.bfloat16)
        a = lax.dot_general(xb, w1, (((1,), (1,)), ((), ())),
                            preferred_element_type=jnp.float32)
        c = lax.dot_general(xb, w3, (((1,), (1,)), ((), ())),
                            preferred_element_type=jnp.float32)
        h = (jax.nn.silu(a) * c).astype(jnp.bfloat16)
        out = lax.dot_general(h, w2, (((1,), (1,)), ((), ())),
                              preferred_element_type=jnp.float32)
        ys_ref[...] = out * wr_ref[0, 0, :][:, None]


def _grouped_mlp(meta, xs, W1, W3, W2, w_pad):
    w3d = w_pad.reshape(NB, 1, BM)
    grid_spec = pltpu.PrefetchScalarGridSpec(
        num_scalar_prefetch=1,
        grid=(NB,),
        in_specs=[
            pl.BlockSpec((BM, D), lambda b, m: (m[2, b], 0)),
            pl.BlockSpec((1, I, D), lambda b, m: (m[0, b], 0, 0)),
            pl.BlockSpec((1, I, D), lambda b, m: (m[0, b], 0, 0)),
            pl.BlockSpec((1, D, I), lambda b, m: (m[0, b], 0, 0)),
            pl.BlockSpec((1, 1, BM), lambda b, m: (b, 0, 0)),
        ],
        out_specs=pl.BlockSpec((BM, D), lambda b, m: (b, 0)),
    )
    return pl.pallas_call(
        _group_body,
        grid_spec=grid_spec,
        out_shape=jax.ShapeDtypeStruct((RPT, D), jnp.float32),
        compiler_params=pltpu.CompilerParams(
            dimension_semantics=("arbitrary",),
            vmem_limit_bytes=120 * 1024 * 1024,
        ),
    )(meta, xs, W1, W3, W2, w3d)


def _shared_body(x_ref, w1_ref, w3_ref, w2_ref, z_ref):
    ic = pl.program_id(1)
    xb = x_ref[...].astype(jnp.bfloat16)
    w1 = w1_ref[...].astype(jnp.bfloat16)
    w3 = w3_ref[...].astype(jnp.bfloat16)
    w2 = w2_ref[...].astype(jnp.bfloat16)
    a = lax.dot_general(xb, w1, (((1,), (1,)), ((), ())),
                        preferred_element_type=jnp.float32)
    c = lax.dot_general(xb, w3, (((1,), (1,)), ((), ())),
                        preferred_element_type=jnp.float32)
    h = (jax.nn.silu(a) * c).astype(jnp.bfloat16)
    out = lax.dot_general(h, w2, (((1,), (1,)), ((), ())),
                          preferred_element_type=jnp.float32)

    @pl.when(ic == 0)
    def _():
        z_ref[...] = out

    @pl.when(ic != 0)
    def _():
        z_ref[...] += out


def _shared_mlp(xf, Ws1, Ws3, Ws2):
    return pl.pallas_call(
        _shared_body,
        grid=(NRB, NIC),
        in_specs=[
            pl.BlockSpec((BMS, D), lambda r, ic: (r, 0)),
            pl.BlockSpec((CSI, D), lambda r, ic: (ic, 0)),
            pl.BlockSpec((CSI, D), lambda r, ic: (ic, 0)),
            pl.BlockSpec((D, CSI), lambda r, ic: (0, ic)),
        ],
        out_specs=pl.BlockSpec((BMS, D), lambda r, ic: (r, 0)),
        out_shape=jax.ShapeDtypeStruct((S, D), jnp.float32),
        compiler_params=pltpu.CompilerParams(
            dimension_semantics=("arbitrary", "arbitrary"),
            vmem_limit_bytes=120 * 1024 * 1024,
        ),
    )(xf, Ws1, Ws3, Ws2)


NC, NS = 2, 16
NW = NC * NS
RPW = RPT // NW
GCH = 32


def _make_row_gather(n_rows, gch, trim=False):
    rpw = n_rows // NW
    nch = rpw // gch

    def body(src_hbm, idx_hbm, *rest):
        if trim:
            tot_hbm, out_hbm, idx_v, tot_v = rest[:4]
            rows0, rows1, rows2, g0, g1, g2, o0, o1, o2 = rest[4:]
        else:
            out_hbm, idx_v = rest[:2]
            rows0, rows1, rows2, g0, g1, g2, o0, o1, o2 = rest[2:]
        wid = lax.axis_index("s") * NC + lax.axis_index("c")
        base = wid * rpw
        pltpu.sync_copy(idx_hbm.at[pl.ds(base, rpw)], idx_v)
        if trim:
            pltpu.sync_copy(tot_hbm, tot_v)
            tot = tot_v[...][0]
        bufs = (rows0, rows1, rows2)
        gsem = (g0, g1, g2)
        osem = (o0, o1, o2)

        def live(c):
            return (base + c * gch) < tot if trim else True

        def guarded(c, fn):
            if trim:
                pl.when(live(c))(fn)
            else:
                fn()

        def fire(c):
            def go():
                pltpu.async_copy(
                    src_hbm.at[idx_v.at[pl.ds(c * gch, gch)]],
                    bufs[c % 3], gsem[c % 3])
            guarded(c, go)

        def wait_g(c):
            def go():
                pltpu.make_async_copy(
                    src_hbm.at[idx_v.at[pl.ds(c * gch, gch)]],
                    bufs[c % 3], gsem[c % 3]).wait()
            guarded(c, go)

        def fire_o(c):
            def go():
                pltpu.async_copy(
                    bufs[c % 3], out_hbm.at[pl.ds(base + c * gch, gch)],
                    osem[c % 3])
            guarded(c, go)

        def wait_o(c):
            def go():
                pltpu.make_async_copy(
                    bufs[c % 3], out_hbm.at[pl.ds(base + c * gch, gch)],
                    osem[c % 3]).wait()
            guarded(c, go)

        fire(0)
        if nch > 1:
            fire(1)
        for c in range(nch):
            wait_g(c)
            fire_o(c)
            if c + 2 < nch:
                if c - 1 >= 0:
                    wait_o(c - 1)
                fire(c + 2)
        for c in range(max(0, nch - 3), nch):
            wait_o(c)

    mesh = plsc.VectorSubcoreMesh(core_axis_name="c", subcore_axis_name="s")
    scratch = [pltpu.VMEM((rpw,), jnp.int32)]
    if trim:
        scratch.append(pltpu.VMEM((16,), jnp.int32))
    scratch += [
        pltpu.VMEM((gch, D), jnp.float32),
        pltpu.VMEM((gch, D), jnp.float32),
        pltpu.VMEM((gch, D), jnp.float32),
        pltpu.SemaphoreType.DMA,
        pltpu.SemaphoreType.DMA,
        pltpu.SemaphoreType.DMA,
        pltpu.SemaphoreType.DMA,
        pltpu.SemaphoreType.DMA,
        pltpu.SemaphoreType.DMA,
    ]
    return pl.kernel(
        body,
        out_type=jax.ShapeDtypeStruct((n_rows, D), jnp.float32),
        mesh=mesh,
        scratch_types=scratch,
    )


def _gather(xf, token_pad, tot16):
    return _make_row_gather(RPT, 16, trim=True)(xf, token_pad, tot16)


BMC = 256


def _add3_body(g0_ref, g1_ref, z_ref, y_ref):
    y_ref[...] = g0_ref[0] + g1_ref[0] + z_ref[...]


def _combine(ys, pos, z):
    g = _make_row_gather(K * S, 16)(ys, pos.reshape(-1))
    g3 = g.reshape(K, S, D)
    return pl.pallas_call(
        _add3_body,
        grid=(S // BMC,),
        in_specs=[
            pl.BlockSpec((1, BMC, D), lambda r: (0, r, 0)),
            pl.BlockSpec((1, BMC, D), lambda r: (1, r, 0)),
            pl.BlockSpec((BMC, D), lambda r: (r, 0)),
        ],
        out_specs=pl.BlockSpec((BMC, D), lambda r: (r, 0)),
        out_shape=jax.ShapeDtypeStruct((S, D), jnp.float32),
        compiler_params=pltpu.CompilerParams(
            dimension_semantics=("arbitrary",),
        ),
    )(g3, g3, z)


def kernel(x, weights, indices, W1, W2, W3, Ws1, Ws2, Ws3):
    xf = x.reshape(-1, D)
    token_pad, w_pad, pos, meta, tot16 = _route(indices, weights)
    xs = _gather(xf, token_pad, tot16)
    z = _shared_mlp(xf, Ws1, Ws3, Ws2)
    ys = _grouped_mlp(meta, xs, W1, W3, W2, w_pad)
    y = _combine(ys, pos, z)
    return y.reshape(x.shape)
